# Initial kernel scaffold; baseline (speedup 1.0000x reference)
import jax
import jax.numpy as jnp
from jax.experimental import pallas as pl


def kernel(x, table):
    raise NotImplementedError("write your pallas kernel here")


# speedup vs baseline: 1.2913x; 1.2913x over previous
import functools
import math

import jax
import jax.numpy as jnp
from jax import lax
from jax.experimental import pallas as pl
from jax.experimental.pallas import tpu as pltpu
from jax.experimental.pallas import tpu_sc as plsc

DIM = 32
SCALE = math.sqrt(float(DIM))

NC, NS = 2, 16
NW = NC * NS
N_TOTAL = 4096 * 200
PER_W = N_TOTAL // NW
CB = 1024
KSUB = CB // 128
CHUNKS = PER_W // CB
XROWS_PER_W = PER_W // 128

_mesh = plsc.VectorSubcoreMesh(core_axis_name="c", subcore_axis_name="s")


@functools.partial(
    pl.kernel,
    out_type=jax.ShapeDtypeStruct((N_TOTAL, DIM), jnp.float32),
    mesh=_mesh,
    compiler_params=pltpu.CompilerParams(use_tc_tiling_on_sc=False),
    scratch_types=[
        pltpu.VMEM((KSUB, 128), jnp.int32),
        pltpu.VMEM((CB, DIM), jnp.float32),
        pltpu.SemaphoreType.DMA,
    ],
)
def _embed_sc(x_hbm, table_hbm, out_hbm, idx_v, rows_v, gsem):
    wid = lax.axis_index("s") * NC + lax.axis_index("c")
    xrow0 = wid * XROWS_PER_W
    out0 = wid * PER_W

    def chunk(ci, carry):
        pltpu.sync_copy(x_hbm.at[pl.ds(xrow0 + ci * KSUB, KSUB)], idx_v)
        copies = [
            pltpu.async_copy(
                table_hbm.at[idx_v.at[j]],
                rows_v.at[pl.ds(j * 128, 128)],
                gsem,
            )
            for j in range(KSUB)
        ]
        for c in copies:
            c.wait()

        def row(i, c):
            rows_v[i, pl.ds(0, 16)] = rows_v[i, pl.ds(0, 16)] * SCALE
            rows_v[i, pl.ds(16, 16)] = rows_v[i, pl.ds(16, 16)] * SCALE
            return c

        lax.fori_loop(0, CB, row, 0)

        pltpu.sync_copy(rows_v, out_hbm.at[pl.ds(out0 + ci * CB, CB)])
        return carry

    lax.fori_loop(0, CHUNKS, chunk, 0)


def kernel(x, table):
    xf = x.reshape(-1).astype(jnp.int32).reshape(N_TOTAL // 128, 128)
    out = _embed_sc(xf, table)
    return out.reshape(4096, 200, DIM)


# speedup vs baseline: 1.4785x; 1.1450x over previous
import functools
import math

import jax
import jax.numpy as jnp
from jax import lax
from jax.experimental import pallas as pl
from jax.experimental.pallas import tpu as pltpu
from jax.experimental.pallas import tpu_sc as plsc

DIM = 32
SCALE = math.sqrt(float(DIM))

NC, NS = 2, 16
NW = NC * NS
N_TOTAL = 4096 * 200
PER_W = N_TOTAL // NW
CB = 640
KSUB = CB // 128
CHUNKS = PER_W // CB
NBUF = 4
LEAD = 2
XR_PER_CHUNK = KSUB

_mesh = plsc.VectorSubcoreMesh(core_axis_name="c", subcore_axis_name="s")


@functools.partial(
    pl.kernel,
    out_type=jax.ShapeDtypeStruct((N_TOTAL, DIM), jnp.float32),
    mesh=_mesh,
    compiler_params=pltpu.CompilerParams(use_tc_tiling_on_sc=False),
    scratch_types=[
        pltpu.VMEM((NBUF, KSUB, 128), jnp.int32),
        pltpu.VMEM((NBUF, CB, DIM), jnp.float32),
    ]
    + [pltpu.SemaphoreType.DMA] * NBUF
    + [pltpu.SemaphoreType.DMA] * NBUF,
)
def _embed_sc(x_hbm, table_hbm, out_hbm, idx_v, rows_v, *sems):
    gsems, osems = sems[:NBUF], sems[NBUF:]
    wid = lax.axis_index("s") * NC + lax.axis_index("c")
    xrow0 = wid * (PER_W // 128)
    out0 = wid * PER_W

    def fire_gather(ci, slot):
        pltpu.sync_copy(
            x_hbm.at[pl.ds(xrow0 + ci * XR_PER_CHUNK, XR_PER_CHUNK)],
            idx_v.at[slot],
        )
        for j in range(KSUB):
            pltpu.async_copy(
                table_hbm.at[idx_v.at[slot].at[j]],
                rows_v.at[slot].at[pl.ds(j * 128, 128)],
                gsems[slot],
            )

    def wait_gather(slot):
        pltpu.make_async_copy(
            table_hbm.at[pl.ds(0, CB)], rows_v.at[slot], gsems[slot]
        ).wait()

    def fire_wb(ci, slot):
        pltpu.async_copy(
            rows_v.at[slot], out_hbm.at[pl.ds(out0 + ci * CB, CB)], osems[slot]
        )

    def wait_wb(slot):
        pltpu.make_async_copy(
            rows_v.at[slot], out_hbm.at[pl.ds(out0, CB)], osems[slot]
        ).wait()

    for ci in range(LEAD):
        fire_gather(ci, ci)

    def outer(g, carry):
        for b in range(NBUF):
            ci = g * NBUF + b
            wait_gather(b)

            fslot = (b + LEAD) % NBUF
            cn = ci + LEAD

            @pl.when(jnp.logical_and(cn < CHUNKS, ci >= NBUF - LEAD))
            def _():
                wait_wb(fslot)

            @pl.when(cn < CHUNKS)
            def _():
                fire_gather(cn, fslot)

            r = rows_v.at[b]

            @plsc.parallel_loop(0, CB, 1, unroll=8)
            def _(i):
                r[i, pl.ds(0, 16)] = r[i, pl.ds(0, 16)] * SCALE
                r[i, pl.ds(16, 16)] = r[i, pl.ds(16, 16)] * SCALE

            fire_wb(ci, b)
        return carry

    lax.fori_loop(0, CHUNKS // NBUF, outer, 0)

    for slot in range(NBUF):
        wait_wb(slot)


def kernel(x, table):
    xf = x.reshape(-1).astype(jnp.int32).reshape(N_TOTAL // 128, 128)
    out = _embed_sc(xf, table)
    return out.reshape(4096, 200, DIM)
